# baseline (speedup 1.0000x reference)
import functools

import jax
import jax.numpy as jnp
from jax import lax
from jax.experimental import pallas as pl
from jax.experimental.pallas import tpu as pltpu
from jax.experimental.pallas import tpu_sc as plsc

B = 4096
EMB = 32
LANES = 128
_CH = 8
_HALF = B // 2

_info = plsc.get_sparse_core_info()
_NC, _NS = _info.num_cores, _info.num_subcores
_NW = _NC * _NS
_MESH = plsc.VectorSubcoreMesh(core_axis_name="c", subcore_axis_name="s")

@functools.cache
def _make_sc_gather(n_items, n_users):
    iw = n_items // LANES if n_items else 0
    uw = n_users // LANES if n_users else 0

    out_type = []
    if n_items:
        out_type.append(jax.ShapeDtypeStruct((EMB, n_items), jnp.float32))
    if n_users:
        out_type.append(jax.ShapeDtypeStruct((EMB, n_users), jnp.float32))

    @functools.partial(
        pl.kernel,
        mesh=_MESH,
        compiler_params=pltpu.CompilerParams(needs_layout_passes=False),
        out_type=out_type,
        scratch_types=[
            pltpu.VMEM((max(n_items, LANES),), jnp.int32),
            pltpu.VMEM((max(n_users, LANES),), jnp.int32),
            pltpu.VMEM((_CH, EMB, LANES), jnp.float32),
            pltpu.VMEM((EMB, LANES), jnp.float32),
            pltpu.SemaphoreType.DMA,
        ],
    )
    def sc_gather(items_hbm, users_hbm, utabT_hbm, itabT_hbm, *refs):
        outs = list(refs[:len(out_type)])
        iidx_all, uidx_all, blk, rows, sem = refs[len(out_type):]
        ioutT_hbm = outs.pop(0) if n_items else None
        uoutT_hbm = outs.pop(0) if n_users else None
        wid = lax.axis_index("s") * _NC + lax.axis_index("c")
        if n_items:
            pltpu.sync_copy(items_hbm, iidx_all)
        if n_users:
            pltpu.sync_copy(users_hbm, uidx_all)

        row16 = lax.iota(jnp.int32, 16)

        def make_round(tabT, idx_all, base):
            def round_body(r, _):
                start = pl.multiple_of(base + r * 16, 16)
                vec = idx_all[pl.ds(start, 16)]
                for h in range(16 // _CH):
                    copies = []
                    for j in range(_CH):
                        u = vec[h * _CH + j]
                        off = pl.multiple_of((u // LANES) * LANES, LANES)
                        copies.append(pltpu.async_copy(
                            tabT.at[:, pl.ds(off, LANES)], blk.at[j], sem))
                    for c in copies:
                        c.wait()
                    for j in range(_CH):
                        col = r * 16 + h * _CH + j
                        c16 = jnp.full((16,), col, dtype=jnp.int32)
                        j16 = jnp.full((16,), j, dtype=jnp.int32)
                        lane16 = jnp.full(
                            (16,), vec[h * _CH + j] % LANES, dtype=jnp.int32)
                        lo = plsc.load_gather(blk, [j16, row16, lane16])
                        hi = plsc.load_gather(blk, [j16, row16 + 16, lane16])
                        plsc.store_scatter(rows, [row16, c16], lo)
                        plsc.store_scatter(rows, [row16 + 16, c16], hi)
                return ()
            return round_body

        if n_items:
            @pl.when(wid < iw)
            def _():
                base = pl.multiple_of(wid * LANES, LANES)
                lax.fori_loop(0, LANES // 16,
                              make_round(itabT_hbm, iidx_all, base), ())
                pltpu.sync_copy(rows, ioutT_hbm.at[:, pl.ds(base, LANES)])

        if n_users:
            @pl.when(wid < uw)
            def _():
                base = pl.multiple_of(wid * LANES, LANES)
                lax.fori_loop(0, LANES // 16,
                              make_round(utabT_hbm, uidx_all, base), ())
                pltpu.sync_copy(rows, uoutT_hbm.at[:, pl.ds(base, LANES)])

    return sc_gather


_BM = 512


def _mm_first_body(a_ref, b_ref, o_ref):
    o_ref[...] = lax.dot_general(
        a_ref[...], b_ref[...],
        (((0,), (0,)), ((), ())),
        preferred_element_type=jnp.float32,
    )


def _mm_second_body(a_ref, b_ref, prev_ref, o_ref):
    del prev_ref
    o_ref[...] = lax.dot_general(
        a_ref[...], b_ref[...],
        (((0,), (0,)), ((), ())),
        preferred_element_type=jnp.float32,
    )


def _score_matmul(uT0, uT1, iT):
    nsteps = _HALF // _BM
    top = pl.pallas_call(
        _mm_first_body,
        grid=(nsteps,),
        in_specs=[
            pl.BlockSpec((EMB, _BM), lambda i: (0, i)),
            pl.BlockSpec((EMB, B), lambda i: (0, 0)),
        ],
        out_specs=pl.BlockSpec((_BM, B), lambda i: (i, 0)),
        out_shape=jax.ShapeDtypeStruct((B, B), jnp.float32),
    )(uT0, iT)
    return pl.pallas_call(
        _mm_second_body,
        grid=(nsteps,),
        in_specs=[
            pl.BlockSpec((EMB, _BM), lambda i: (0, i)),
            pl.BlockSpec((EMB, B), lambda i: (0, 0)),
            pl.BlockSpec(memory_space=pl.ANY),
        ],
        out_specs=pl.BlockSpec((_BM, B), lambda i: (i + nsteps, 0)),
        out_shape=jax.ShapeDtypeStruct((B, B), jnp.float32),
        input_output_aliases={2: 0},
    )(uT1, iT, top)


def kernel(users, pos_items, user_table, item_table):
    utT = user_table.T
    itT = item_table.T
    zeros_i = jnp.zeros((LANES,), jnp.int32)
    iT, uT0 = _make_sc_gather(B, _HALF)(pos_items, users[:_HALF], utT, itT)
    u1_idx, _ = lax.optimization_barrier((users[_HALF:], uT0))
    (uT1,) = _make_sc_gather(0, _HALF)(zeros_i, u1_idx, utT, itT)
    return _score_matmul(uT0, uT1, iT)


# speedup vs baseline: 1.4006x; 1.4006x over previous
import functools

import jax
import jax.numpy as jnp
from jax import lax
from jax.experimental import pallas as pl
from jax.experimental.pallas import tpu as pltpu
from jax.experimental.pallas import tpu_sc as plsc

B = 4096
EMB = 32
LANES = 128
_CH = 8


def _make_sc_gather():
    info = plsc.get_sparse_core_info()
    nc, ns = info.num_cores, info.num_subcores
    nw = nc * ns
    b_per_w = B // nw
    rounds = b_per_w // 16

    mesh = plsc.VectorSubcoreMesh(core_axis_name="c", subcore_axis_name="s")

    @functools.partial(
        pl.kernel,
        mesh=mesh,
        compiler_params=pltpu.CompilerParams(needs_layout_passes=False),
        out_type=[
            jax.ShapeDtypeStruct((EMB, B), jnp.float32),
            jax.ShapeDtypeStruct((EMB, B), jnp.float32),
        ],
        scratch_types=[
            pltpu.VMEM((B,), jnp.int32),
            pltpu.VMEM((B,), jnp.int32),
            pltpu.VMEM((_CH, EMB, LANES), jnp.float32),
            pltpu.VMEM((_CH, EMB, LANES), jnp.float32),
            pltpu.VMEM((EMB, b_per_w), jnp.float32),
            pltpu.VMEM((EMB, b_per_w), jnp.float32),
            pltpu.SemaphoreType.DMA,
            pltpu.SemaphoreType.DMA,
        ],
    )
    def sc_gather(users_hbm, items_hbm, utabT_hbm, itabT_hbm,
                  uoutT_hbm, ioutT_hbm,
                  uidx_all, iidx_all, ublk, iblk, urows, irows, usem, isem):
        wid = lax.axis_index("s") * nc + lax.axis_index("c")
        base = pl.multiple_of(wid * b_per_w, b_per_w)
        pltpu.sync_copy(users_hbm, uidx_all)
        pltpu.sync_copy(items_hbm, iidx_all)

        row16 = lax.iota(jnp.int32, 16)

        def round_body(r, _):
            start = pl.multiple_of(base + r * 16, 16)
            uvec = uidx_all[pl.ds(start, 16)]
            ivec = iidx_all[pl.ds(start, 16)]
            for h in range(16 // _CH):
                copies = []
                for j in range(_CH):
                    u = uvec[h * _CH + j]
                    uoff = pl.multiple_of((u // LANES) * LANES, LANES)
                    copies.append(pltpu.async_copy(
                        utabT_hbm.at[:, pl.ds(uoff, LANES)], ublk.at[j], usem))
                    i = ivec[h * _CH + j]
                    ioff = pl.multiple_of((i // LANES) * LANES, LANES)
                    copies.append(pltpu.async_copy(
                        itabT_hbm.at[:, pl.ds(ioff, LANES)], iblk.at[j], isem))
                for c in copies:
                    c.wait()
                for j in range(_CH):
                    b16 = jnp.full((16,), r * 16 + h * _CH + j, dtype=jnp.int32)
                    j16 = jnp.full((16,), j, dtype=jnp.int32)
                    for vec, blk, rows in ((uvec, ublk, urows),
                                           (ivec, iblk, irows)):
                        lane16 = jnp.full(
                            (16,), vec[h * _CH + j] % LANES, dtype=jnp.int32)
                        lo = plsc.load_gather(blk, [j16, row16, lane16])
                        hi = plsc.load_gather(blk, [j16, row16 + 16, lane16])
                        plsc.store_scatter(rows, [row16, b16], lo)
                        plsc.store_scatter(rows, [row16 + 16, b16], hi)
            return ()

        lax.fori_loop(0, rounds, round_body, ())

        pltpu.sync_copy(urows, uoutT_hbm.at[:, pl.ds(base, b_per_w)])
        pltpu.sync_copy(irows, ioutT_hbm.at[:, pl.ds(base, b_per_w)])

    return sc_gather


_sc_gather = _make_sc_gather()


_BM = 512


def _matmul_body(a_ref, b_ref, o_ref):
    o_ref[...] = lax.dot_general(
        a_ref[...], b_ref[...],
        (((0,), (0,)), ((), ())),
        preferred_element_type=jnp.float32,
    )


def _score_matmul(uT, iT):
    grid = (B // _BM,)
    return pl.pallas_call(
        _matmul_body,
        grid=grid,
        in_specs=[
            pl.BlockSpec((EMB, _BM), lambda i: (0, i)),
            pl.BlockSpec((EMB, B), lambda i: (0, 0)),
        ],
        out_specs=pl.BlockSpec((_BM, B), lambda i: (i, 0)),
        out_shape=jax.ShapeDtypeStruct((B, B), jnp.float32),
    )(uT, iT)


def kernel(users, pos_items, user_table, item_table):
    uT, iT = _sc_gather(users, pos_items, user_table.T, item_table.T)
    return _score_matmul(uT, iT)


# speedup vs baseline: 1.4518x; 1.0365x over previous
import functools

import jax
import jax.numpy as jnp
from jax import lax
from jax.experimental import pallas as pl
from jax.experimental.pallas import tpu as pltpu
from jax.experimental.pallas import tpu_sc as plsc

B = 4096
EMB = 32
LANES = 128
_QC = 4


def _make_sc_gather():
    info = plsc.get_sparse_core_info()
    nc, ns = info.num_cores, info.num_subcores
    nw = nc * ns
    b_per_w = B // nw
    rounds = b_per_w // 16

    mesh = plsc.VectorSubcoreMesh(core_axis_name="c", subcore_axis_name="s")

    @functools.partial(
        pl.kernel,
        mesh=mesh,
        compiler_params=pltpu.CompilerParams(needs_layout_passes=False),
        out_type=[
            jax.ShapeDtypeStruct((EMB, B), jnp.float32),
            jax.ShapeDtypeStruct((EMB, B), jnp.float32),
        ],
        scratch_types=[
            pltpu.VMEM((B,), jnp.int32),
            pltpu.VMEM((B,), jnp.int32),
            pltpu.VMEM((2 * _QC, EMB, LANES), jnp.float32),
            pltpu.VMEM((2 * _QC, EMB, LANES), jnp.float32),
            pltpu.VMEM((EMB, b_per_w), jnp.float32),
            pltpu.VMEM((EMB, b_per_w), jnp.float32),
            pltpu.SemaphoreType.DMA,
            pltpu.SemaphoreType.DMA,
            pltpu.SemaphoreType.DMA,
            pltpu.SemaphoreType.DMA,
        ],
    )
    def sc_gather(users_hbm, items_hbm, utabT_hbm, itabT_hbm,
                  uoutT_hbm, ioutT_hbm,
                  uidx_all, iidx_all, ublk, iblk, urows, irows,
                  usem0, usem1, isem0, isem1):
        usems, isems = (usem0, usem1), (isem0, isem1)
        wid = lax.axis_index("s") * nc + lax.axis_index("c")
        base = pl.multiple_of(wid * b_per_w, b_per_w)
        pltpu.sync_copy(users_hbm, uidx_all)
        pltpu.sync_copy(items_hbm, iidx_all)

        row16 = lax.iota(jnp.int32, 16)

        nq = 16 // _QC

        def round_body(r, _):
            start = pl.multiple_of(base + r * 16, 16)
            uvec = uidx_all[pl.ds(start, 16)]
            ivec = iidx_all[pl.ds(start, 16)]

            def fire(q):
                slot = (q % 2) * _QC
                copies = []
                for j in range(_QC):
                    u = uvec[q * _QC + j]
                    uoff = pl.multiple_of((u // LANES) * LANES, LANES)
                    copies.append(pltpu.async_copy(
                        utabT_hbm.at[:, pl.ds(uoff, LANES)],
                        ublk.at[slot + j], usems[q % 2]))
                    i = ivec[q * _QC + j]
                    ioff = pl.multiple_of((i // LANES) * LANES, LANES)
                    copies.append(pltpu.async_copy(
                        itabT_hbm.at[:, pl.ds(ioff, LANES)],
                        iblk.at[slot + j], isems[q % 2]))
                return copies

            def extract(q):
                slot = (q % 2) * _QC
                for j in range(_QC):
                    b16 = jnp.full((16,), r * 16 + q * _QC + j, dtype=jnp.int32)
                    j16 = jnp.full((16,), slot + j, dtype=jnp.int32)
                    for vec, blk, rows in ((uvec, ublk, urows),
                                           (ivec, iblk, irows)):
                        lane16 = jnp.full(
                            (16,), vec[q * _QC + j] % LANES, dtype=jnp.int32)
                        lo = plsc.load_gather(blk, [j16, row16, lane16])
                        hi = plsc.load_gather(blk, [j16, row16 + 16, lane16])
                        plsc.store_scatter(rows, [row16, b16], lo)
                        plsc.store_scatter(rows, [row16 + 16, b16], hi)

            inflight = fire(0)
            for q in range(nq):
                nxt = fire(q + 1) if q + 1 < nq else []
                for c in inflight:
                    c.wait()
                extract(q)
                inflight = nxt
            return ()

        lax.fori_loop(0, rounds, round_body, ())

        pltpu.sync_copy(urows, uoutT_hbm.at[:, pl.ds(base, b_per_w)])
        pltpu.sync_copy(irows, ioutT_hbm.at[:, pl.ds(base, b_per_w)])

    return sc_gather


_sc_gather = _make_sc_gather()


_BM = 512


def _matmul_body(a_ref, b_ref, o_ref):
    o_ref[...] = lax.dot_general(
        a_ref[...], b_ref[...],
        (((0,), (0,)), ((), ())),
        preferred_element_type=jnp.float32,
    )


def _score_matmul(uT, iT):
    grid = (B // _BM,)
    return pl.pallas_call(
        _matmul_body,
        grid=grid,
        in_specs=[
            pl.BlockSpec((EMB, _BM), lambda i: (0, i)),
            pl.BlockSpec((EMB, B), lambda i: (0, 0)),
        ],
        out_specs=pl.BlockSpec((_BM, B), lambda i: (i, 0)),
        out_shape=jax.ShapeDtypeStruct((B, B), jnp.float32),
    )(uT, iT)


def kernel(users, pos_items, user_table, item_table):
    uT, iT = _sc_gather(users, pos_items, user_table.T, item_table.T)
    return _score_matmul(uT, iT)
